# Initial kernel scaffold; baseline (speedup 1.0000x reference)
import jax
import jax.numpy as jnp
from jax.experimental import pallas as pl


def kernel(semantic, sem_target):
    raise NotImplementedError("write your pallas kernel here")


# speedup vs baseline: 34.8812x; 34.8812x over previous
import functools

import numpy as np
import jax
import jax.numpy as jnp
from jax import lax
from jax.experimental import pallas as pl
from jax.experimental.pallas import tpu as pltpu
from jax.experimental.pallas import tpu_sc as plsc

C = 18
N = 200 * 200 * 16
NH = N // 2
K = 8192
HIST = 4 * K
NW = 32
WPW = (C * NH) // NW
CHUNK = 20000
NCHUNK = WPW // CHUNK
VB = 5120
HB = VB // 2
NBLK = N // VB


def _pass1_body(sem_ref, tgt_ref, words_ref, ce_ref):
    l = sem_ref[...]
    t = tgt_ref[...].reshape(1, VB)
    m = jnp.max(l, axis=0, keepdims=True)
    ex = jnp.exp(l - m)
    s = jnp.sum(ex, axis=0, keepdims=True)
    p = ex / s
    ci = lax.broadcasted_iota(jnp.int32, (C, VB), 0)
    fg = ci == t
    err = jnp.where(fg, 1.0 - p, p)
    qa = jnp.minimum((err * K).astype(jnp.int32), K - 1)
    key = jnp.where(fg, qa + K, qa)
    ka = key[:, :HB]
    kb = key[:, HB:]
    words_ref[...] = ka | (kb << 16)
    lse = m + jnp.log(s)
    lt = jnp.sum(jnp.where(fg, l, 0.0), axis=0, keepdims=True)
    ce_blk = jnp.sum(lse - lt)

    @pl.when(pl.program_id(0) == 0)
    def _():
        ce_ref[...] = jnp.zeros((1, 1), jnp.float32)

    ce_ref[...] += ce_blk


def _sc_body(words_hbm, out_hbm, hist_v, buf_v):
    cid = lax.axis_index("c")
    sid = lax.axis_index("s")
    wid = sid * 2 + cid
    wbase = wid * WPW
    c0 = lax.shift_right_logical(wid * 9, 4)
    bound = (c0 + 1) * NH

    zeros = jnp.zeros((16,), jnp.int32)
    ones = jnp.ones((16,), jnp.int32)

    def zbody(i, carry):
        hist_v[pl.ds(i * 16, 16)] = zeros
        return carry

    lax.fori_loop(0, HIST // 16, zbody, 0)

    def chunk(j, carry):
        start = wbase + j * CHUNK
        is1 = (start >= bound).astype(jnp.int32)
        pltpu.sync_copy(words_hbm.at[pl.ds(start, CHUNK)], buf_v)
        off = is1 * (2 * K)

        def vec(i, c2):
            w = buf_v[pl.ds(i * 16, 16)]
            lo = (w & 0xFFFF) + off
            hi = lax.shift_right_logical(w, 16) + off
            plsc.addupdate_scatter(hist_v, [lo], ones)
            plsc.addupdate_scatter(hist_v, [hi], ones)
            return c2

        lax.fori_loop(0, CHUNK // 16, vec, 0)
        return carry

    lax.fori_loop(0, NCHUNK, chunk, 0)
    pltpu.sync_copy(hist_v, out_hbm.at[pl.ds(wid * HIST, HIST)])


def _suffix_sum(x):
    d = 1
    while d < K:
        pad = jnp.zeros((C, d), jnp.float32)
        x = x + jnp.concatenate([x[:, d:], pad], axis=1)
        d *= 2
    return x


def _pass2_body(h_ref, mm_ref, ce_ref, tot_ref, ceo_ref, lov_ref):
    hf = h_ref[...]
    mm = mm_ref[...]
    merged = jnp.dot(mm, hf, preferred_element_type=jnp.float32)
    fh = merged[:, K:]
    mh = merged[:, :K] + fh
    sm = _suffix_sum(mh)
    sf = _suffix_sum(fh)
    n = sf[:, 0:1]
    s_abv = sm - mh
    f_lo = sf - fh
    j_hi = 1.0 - (n - sf) / (n + s_abv + mh - sf + 1e-6)
    j_lo = 1.0 - (n - f_lo) / (n + s_abv - f_lo + 1e-6)
    j_lo = jnp.where(s_abv > 0, j_lo, 0.0)
    kk = lax.broadcasted_iota(jnp.int32, (C, K), 1).astype(jnp.float32)
    e_hat = (kk + 0.5) * (1.0 / K)
    contrib = jnp.where(mh > 0, e_hat * (j_hi - j_lo), 0.0)
    losses = jnp.sum(contrib, axis=1, keepdims=True)
    pres = (n > 0).astype(jnp.float32)
    npres = jnp.sum(pres)
    lov = jnp.sum(losses * pres) / jnp.maximum(npres, 1.0)
    lov = jnp.where(npres > 0, lov, 0.0)
    ce = ce_ref[0, 0] * (1.0 / N)
    tot_ref[...] = jnp.full((1, 1), ce + 0.5 * lov, jnp.float32)
    ceo_ref[...] = jnp.full((1, 1), ce, jnp.float32)
    lov_ref[...] = jnp.full((1, 1), lov, jnp.float32)


def _merge_matrix():
    m = np.zeros((C, 2 * NW), np.float32)
    for w in range(NW):
        c0 = (9 * w) >> 4
        m[c0, 2 * w] = 1.0
        if c0 + 1 < C:
            m[c0 + 1, 2 * w + 1] = 1.0
    return jnp.asarray(m)


_pass1 = pl.pallas_call(
    _pass1_body,
    grid=(NBLK,),
    in_specs=[
        pl.BlockSpec((C, VB), lambda i: (0, i)),
        pl.BlockSpec((1, 1, VB), lambda i: (i, 0, 0)),
    ],
    out_specs=[
        pl.BlockSpec((C, HB), lambda i: (0, i)),
        pl.BlockSpec((1, 1), lambda i: (0, 0)),
    ],
    out_shape=[
        jax.ShapeDtypeStruct((C, NH), jnp.int32),
        jax.ShapeDtypeStruct((1, 1), jnp.float32),
    ],
)


@functools.cache
def _sc_hist():
    return pl.kernel(
        _sc_body,
        out_type=jax.ShapeDtypeStruct((NW * HIST,), jnp.int32),
        mesh=plsc.VectorSubcoreMesh(
            core_axis_name="c", subcore_axis_name="s", num_cores=2),
        scratch_types=[
            pltpu.VMEM((HIST,), jnp.int32),
            pltpu.VMEM((CHUNK,), jnp.int32),
        ],
        compiler_params=pltpu.CompilerParams(needs_layout_passes=False),
    )


_pass2 = pl.pallas_call(
    _pass2_body,
    out_shape=[
        jax.ShapeDtypeStruct((1, 1), jnp.float32),
        jax.ShapeDtypeStruct((1, 1), jnp.float32),
        jax.ShapeDtypeStruct((1, 1), jnp.float32),
    ],
)


def kernel(semantic, sem_target):
    sem2 = semantic.reshape(C, N)
    tgt3 = sem_target.reshape(NBLK, 1, VB).astype(jnp.int32)
    words, ce = _pass1(sem2, tgt3)
    hist = _sc_hist()(words.reshape(C * NH))
    hf = hist.reshape(2 * NW, 2 * K).astype(jnp.float32)
    tot, ceo, lov = _pass2(hf, _merge_matrix(), ce)
    return tot.reshape(()), ceo.reshape(()), lov.reshape(())


# speedup vs baseline: 36.8665x; 1.0569x over previous
import functools

import numpy as np
import jax
import jax.numpy as jnp
from jax import lax
from jax.experimental import pallas as pl
from jax.experimental.pallas import tpu as pltpu
from jax.experimental.pallas import tpu_sc as plsc

C = 18
N = 200 * 200 * 16
NH = N // 2
K = 8192
HIST = 4 * K
NW = 32
WPW = (C * NH) // NW
CHUNK = 10000
NCHUNK = WPW // CHUNK
VB = 5120
HB = VB // 2
NBLK = N // VB


def _pass1_body(sem_ref, tgt_ref, words_ref, ce_ref):
    l = sem_ref[...]
    t = tgt_ref[...].reshape(1, VB)
    m = jnp.max(l, axis=0, keepdims=True)
    ex = jnp.exp(l - m)
    s = jnp.sum(ex, axis=0, keepdims=True)
    p = ex / s
    ci = lax.broadcasted_iota(jnp.int32, (C, VB), 0)
    fg = ci == t
    err = jnp.where(fg, 1.0 - p, p)
    qa = jnp.minimum((err * K).astype(jnp.int32), K - 1)
    key = jnp.where(fg, qa + K, qa)
    ka = key[:, :HB]
    kb = key[:, HB:]
    words_ref[...] = ka | (kb << 16)
    lse = m + jnp.log(s)
    lt = jnp.sum(jnp.where(fg, l, 0.0), axis=0, keepdims=True)
    ce_blk = jnp.sum(lse - lt)

    @pl.when(pl.program_id(0) == 0)
    def _():
        ce_ref[...] = jnp.zeros((1, 1), jnp.float32)

    ce_ref[...] += ce_blk


def _sc_body(words_hbm, out_hbm, hist_v, buf0, buf1, sem0, sem1):
    cid = lax.axis_index("c")
    sid = lax.axis_index("s")
    wid = sid * 2 + cid
    wbase = wid * WPW
    c0 = lax.shift_right_logical(wid * 9, 4)
    bound = (c0 + 1) * NH

    zeros = jnp.zeros((16,), jnp.int32)
    ones = jnp.ones((16,), jnp.int32)

    def zbody(i, carry):
        hist_v[pl.ds(i * 16, 16)] = zeros
        return carry

    lax.fori_loop(0, HIST // 16, zbody, 0, unroll=8)

    def chunk_src(j):
        return words_hbm.at[pl.ds(wbase + j * CHUNK, CHUNK)]

    def process(j, buf):
        start = wbase + j * CHUNK
        off = (start >= bound).astype(jnp.int32) * (2 * K)

        def vec(i, c2):
            w = buf[pl.ds(i * 16, 16)]
            lo = (w & 0xFFFF) + off
            hi = lax.shift_right_logical(w, 16) + off
            plsc.addupdate_scatter(hist_v, [lo], ones)
            plsc.addupdate_scatter(hist_v, [hi], ones)
            return c2

        lax.fori_loop(0, CHUNK // 16, vec, 0, unroll=8)

    pltpu.async_copy(chunk_src(0), buf0, sem0)

    def pair(jp, carry):
        j0 = 2 * jp
        pltpu.async_copy(chunk_src(j0 + 1), buf1, sem1)
        pltpu.make_async_copy(chunk_src(j0), buf0, sem0).wait()
        process(j0, buf0)

        @pl.when(j0 + 2 < NCHUNK)
        def _():
            pltpu.async_copy(chunk_src(j0 + 2), buf0, sem0)

        pltpu.make_async_copy(chunk_src(j0 + 1), buf1, sem1).wait()
        process(j0 + 1, buf1)
        return carry

    lax.fori_loop(0, NCHUNK // 2, pair, 0)
    pltpu.sync_copy(hist_v, out_hbm.at[pl.ds(wid * HIST, HIST)])


def _suffix_sum(x):
    d = 1
    while d < K:
        pad = jnp.zeros((C, d), jnp.float32)
        x = x + jnp.concatenate([x[:, d:], pad], axis=1)
        d *= 2
    return x


def _pass2_body(h_ref, mm_ref, ce_ref, tot_ref, ceo_ref, lov_ref):
    hf = h_ref[...]
    mm = mm_ref[...]
    merged = jnp.dot(mm, hf, preferred_element_type=jnp.float32)
    fh = merged[:, K:]
    mh = merged[:, :K] + fh
    sm = _suffix_sum(mh)
    sf = _suffix_sum(fh)
    n = sf[:, 0:1]
    s_abv = sm - mh
    f_lo = sf - fh
    j_hi = 1.0 - (n - sf) / (n + s_abv + mh - sf + 1e-6)
    j_lo = 1.0 - (n - f_lo) / (n + s_abv - f_lo + 1e-6)
    j_lo = jnp.where(s_abv > 0, j_lo, 0.0)
    kk = lax.broadcasted_iota(jnp.int32, (C, K), 1).astype(jnp.float32)
    e_hat = (kk + 0.5) * (1.0 / K)
    contrib = jnp.where(mh > 0, e_hat * (j_hi - j_lo), 0.0)
    losses = jnp.sum(contrib, axis=1, keepdims=True)
    pres = (n > 0).astype(jnp.float32)
    npres = jnp.sum(pres)
    lov = jnp.sum(losses * pres) / jnp.maximum(npres, 1.0)
    lov = jnp.where(npres > 0, lov, 0.0)
    ce = ce_ref[0, 0] * (1.0 / N)
    tot_ref[...] = jnp.full((1, 1), ce + 0.5 * lov, jnp.float32)
    ceo_ref[...] = jnp.full((1, 1), ce, jnp.float32)
    lov_ref[...] = jnp.full((1, 1), lov, jnp.float32)


def _merge_matrix():
    m = np.zeros((C, 2 * NW), np.float32)
    for w in range(NW):
        c0 = (9 * w) >> 4
        m[c0, 2 * w] = 1.0
        if c0 + 1 < C:
            m[c0 + 1, 2 * w + 1] = 1.0
    return jnp.asarray(m)


_pass1 = pl.pallas_call(
    _pass1_body,
    grid=(NBLK,),
    in_specs=[
        pl.BlockSpec((C, VB), lambda i: (0, i)),
        pl.BlockSpec((1, 1, VB), lambda i: (i, 0, 0)),
    ],
    out_specs=[
        pl.BlockSpec((C, HB), lambda i: (0, i)),
        pl.BlockSpec((1, 1), lambda i: (0, 0)),
    ],
    out_shape=[
        jax.ShapeDtypeStruct((C, NH), jnp.int32),
        jax.ShapeDtypeStruct((1, 1), jnp.float32),
    ],
)


@functools.cache
def _sc_hist():
    return pl.kernel(
        _sc_body,
        out_type=jax.ShapeDtypeStruct((NW * HIST,), jnp.int32),
        mesh=plsc.VectorSubcoreMesh(
            core_axis_name="c", subcore_axis_name="s", num_cores=2),
        scratch_types=[
            pltpu.VMEM((HIST,), jnp.int32),
            pltpu.VMEM((CHUNK,), jnp.int32),
            pltpu.VMEM((CHUNK,), jnp.int32),
            pltpu.SemaphoreType.DMA,
            pltpu.SemaphoreType.DMA,
        ],
        compiler_params=pltpu.CompilerParams(needs_layout_passes=False),
    )


_pass2 = pl.pallas_call(
    _pass2_body,
    out_shape=[
        jax.ShapeDtypeStruct((1, 1), jnp.float32),
        jax.ShapeDtypeStruct((1, 1), jnp.float32),
        jax.ShapeDtypeStruct((1, 1), jnp.float32),
    ],
)


def kernel(semantic, sem_target):
    sem2 = semantic.reshape(C, N)
    tgt3 = sem_target.reshape(NBLK, 1, VB).astype(jnp.int32)
    words, ce = _pass1(sem2, tgt3)
    hist = _sc_hist()(words.reshape(C * NH))
    hf = hist.reshape(2 * NW, 2 * K).astype(jnp.float32)
    tot, ceo, lov = _pass2(hf, _merge_matrix(), ce)
    return tot.reshape(()), ceo.reshape(()), lov.reshape(())


# speedup vs baseline: 52.2472x; 1.4172x over previous
import functools

import numpy as np
import jax
import jax.numpy as jnp
from jax import lax
from jax.experimental import pallas as pl
from jax.experimental.pallas import tpu as pltpu
from jax.experimental.pallas import tpu_sc as plsc

C = 18
N = 200 * 200 * 16
NH = N // 2
K = 8192
HIST = 4 * K
NW = 32
WPW = (C * NH) // NW
CHUNK = 10000
NCHUNK = WPW // CHUNK
VB = 5120
HB = VB // 2
NBLK = N // VB


def _pass1_body(sem_ref, tgt_ref, words_ref, ce_ref):
    l = sem_ref[...]
    t = tgt_ref[...].reshape(1, VB)
    m = jnp.max(l, axis=0, keepdims=True)
    ex = jnp.exp(l - m)
    s = jnp.sum(ex, axis=0, keepdims=True)
    p = ex / s
    ci = lax.broadcasted_iota(jnp.int32, (C, VB), 0)
    fg = ci == t
    err = jnp.where(fg, 1.0 - p, p)
    qa = jnp.minimum((err * K).astype(jnp.int32), K - 1)
    key = jnp.where(fg, qa + K, qa)
    ka = key[:, :HB]
    kb = key[:, HB:]
    words_ref[...] = ka | (kb << 16)
    lse = m + jnp.log(s)
    lt = jnp.sum(jnp.where(fg, l, 0.0), axis=0, keepdims=True)
    ce_blk = jnp.sum(lse - lt)

    @pl.when(pl.program_id(0) == 0)
    def _():
        ce_ref[...] = jnp.zeros((1, 1), jnp.float32)

    ce_ref[...] += ce_blk


def _sc_body(words_hbm, out_hbm, hist_v, buf0, buf1, sem0, sem1):
    cid = lax.axis_index("c")
    sid = lax.axis_index("s")
    wid = sid * 2 + cid
    wbase = wid * WPW
    c0 = lax.shift_right_logical(wid * 9, 4)
    bound = (c0 + 1) * NH

    zeros = jnp.zeros((16,), jnp.int32)
    ones = jnp.ones((16,), jnp.int32)

    def zbody(i, carry):
        hist_v[pl.ds(i * 16, 16)] = zeros
        return carry

    lax.fori_loop(0, HIST // 16, zbody, 0, unroll=8)

    def chunk_src(j):
        return words_hbm.at[pl.ds(wbase + j * CHUNK, CHUNK)]

    def process(j, buf):
        start = wbase + j * CHUNK
        off = (start >= bound).astype(jnp.int32) * (2 * K)

        def vec(i, c2):
            w = buf[pl.ds(i * 16, 16)]
            lo = (w & 0xFFFF) + off
            hi = lax.shift_right_logical(w, 16) + off
            plsc.addupdate_scatter(hist_v, [lo], ones)
            plsc.addupdate_scatter(hist_v, [hi], ones)
            return c2

        lax.fori_loop(0, CHUNK // 16, vec, 0, unroll=8)

    pltpu.async_copy(chunk_src(0), buf0, sem0)

    def pair(jp, carry):
        j0 = 2 * jp
        pltpu.async_copy(chunk_src(j0 + 1), buf1, sem1)
        pltpu.make_async_copy(chunk_src(j0), buf0, sem0).wait()
        process(j0, buf0)

        @pl.when(j0 + 2 < NCHUNK)
        def _():
            pltpu.async_copy(chunk_src(j0 + 2), buf0, sem0)

        pltpu.make_async_copy(chunk_src(j0 + 1), buf1, sem1).wait()
        process(j0 + 1, buf1)
        return carry

    lax.fori_loop(0, NCHUNK // 2, pair, 0)
    pltpu.sync_copy(hist_v, out_hbm.at[pl.ds(wid * HIST, HIST)])


def _suffix_sum(x):
    d = 1
    while d < K:
        pad = jnp.zeros((C, d), jnp.float32)
        x = x + jnp.concatenate([x[:, d:], pad], axis=1)
        d *= 2
    return x


def _pass2_body(h_ref, mm_ref, ce_ref, tot_ref, ceo_ref, lov_ref):
    hf = h_ref[...]
    mm = mm_ref[...]
    merged = jnp.dot(mm, hf, preferred_element_type=jnp.float32)
    fh = merged[:, K:]
    mh = merged[:, :K] + fh
    sm = _suffix_sum(mh)
    sf = _suffix_sum(fh)
    n = sf[:, 0:1]
    s_abv = sm - mh
    f_lo = sf - fh
    j_hi = 1.0 - (n - sf) / (n + s_abv + mh - sf + 1e-6)
    j_lo = 1.0 - (n - f_lo) / (n + s_abv - f_lo + 1e-6)
    j_lo = jnp.where(s_abv > 0, j_lo, 0.0)
    kk = lax.broadcasted_iota(jnp.int32, (C, K), 1).astype(jnp.float32)
    e_hat = (kk + 0.5) * (1.0 / K)
    contrib = jnp.where(mh > 0, e_hat * (j_hi - j_lo), 0.0)
    losses = jnp.sum(contrib, axis=1, keepdims=True)
    pres = (n > 0).astype(jnp.float32)
    npres = jnp.sum(pres)
    lov = jnp.sum(losses * pres) / jnp.maximum(npres, 1.0)
    lov = jnp.where(npres > 0, lov, 0.0)
    ce = ce_ref[0, 0] * (1.0 / N)
    tot_ref[...] = jnp.full((1, 1), ce + 0.5 * lov, jnp.float32)
    ceo_ref[...] = jnp.full((1, 1), ce, jnp.float32)
    lov_ref[...] = jnp.full((1, 1), lov, jnp.float32)


def _merge_matrix():
    m = np.zeros((C, 2 * NW), np.float32)
    for w in range(NW):
        c0 = (9 * w) >> 4
        m[c0, 2 * w] = 1.0
        if c0 + 1 < C:
            m[c0 + 1, 2 * w + 1] = 1.0
    return jnp.asarray(m)


_pass1 = pl.pallas_call(
    _pass1_body,
    grid=(NBLK,),
    in_specs=[
        pl.BlockSpec((C, VB), lambda i: (0, i)),
        pl.BlockSpec((1, 1, VB), lambda i: (i, 0, 0)),
    ],
    out_specs=[
        pl.BlockSpec((C, HB), lambda i: (0, i)),
        pl.BlockSpec((1, 1), lambda i: (0, 0)),
    ],
    out_shape=[
        jax.ShapeDtypeStruct((C, NH), jnp.int32),
        jax.ShapeDtypeStruct((1, 1), jnp.float32),
    ],
)


@functools.cache
def _sc_hist():
    return pl.kernel(
        _sc_body,
        out_type=jax.ShapeDtypeStruct((NW * HIST,), jnp.int32),
        mesh=plsc.VectorSubcoreMesh(
            core_axis_name="c", subcore_axis_name="s", num_cores=2),
        scratch_types=[
            pltpu.VMEM((HIST,), jnp.int32),
            pltpu.VMEM((CHUNK,), jnp.int32),
            pltpu.VMEM((CHUNK,), jnp.int32),
            pltpu.SemaphoreType.DMA,
            pltpu.SemaphoreType.DMA,
        ],
        compiler_params=pltpu.CompilerParams(needs_layout_passes=False),
    )


_pass2 = pl.pallas_call(
    _pass2_body,
    out_shape=[
        jax.ShapeDtypeStruct((1, 1), jnp.float32),
        jax.ShapeDtypeStruct((1, 1), jnp.float32),
        jax.ShapeDtypeStruct((1, 1), jnp.float32),
    ],
)


def kernel(semantic, sem_target):
    sem2 = jnp.transpose(semantic, (0, 1, 2, 4, 3)).reshape(C, N)
    tgt3 = jnp.transpose(sem_target, (0, 1, 3, 2)).reshape(NBLK, 1, VB)
    tgt3 = tgt3.astype(jnp.int32)
    words, ce = _pass1(sem2, tgt3)
    hist = _sc_hist()(words.reshape(C * NH))
    hf = hist.reshape(2 * NW, 2 * K).astype(jnp.float32)
    tot, ceo, lov = _pass2(hf, _merge_matrix(), ce)
    return tot.reshape(()), ceo.reshape(()), lov.reshape(())


# speedup vs baseline: 56.1706x; 1.0751x over previous
import functools

import numpy as np
import jax
import jax.numpy as jnp
from jax import lax
from jax.experimental import pallas as pl
from jax.experimental.pallas import tpu as pltpu
from jax.experimental.pallas import tpu_sc as plsc

C = 18
N = 200 * 200 * 16
NH = N // 2
K = 8192
HIST = 4 * K
NW = 32
WPW = (C * NH) // NW
CHUNK = 10000
NCHUNK = WPW // CHUNK
VB = 12800
HB = VB // 2
NBLK = N // VB


def _pass1_body(sem_ref, tgt_ref, words_ref, ce_ref):
    l = sem_ref[...]
    t = tgt_ref[...].reshape(1, VB)
    m = jnp.max(l, axis=0, keepdims=True)
    ex = jnp.exp(l - m)
    s = jnp.sum(ex, axis=0, keepdims=True)
    p = ex / s
    ci = lax.broadcasted_iota(jnp.int32, (C, VB), 0)
    fg = ci == t
    err = jnp.where(fg, 1.0 - p, p)
    qa = jnp.minimum((err * K).astype(jnp.int32), K - 1)
    key = jnp.where(fg, qa + K, qa)
    ka = key[:, :HB]
    kb = key[:, HB:]
    words_ref[...] = ka | (kb << 16)
    lse = m + jnp.log(s)
    lt = jnp.sum(jnp.where(fg, l, 0.0), axis=0, keepdims=True)
    ce_blk = jnp.sum(lse - lt)

    @pl.when(pl.program_id(0) == 0)
    def _():
        ce_ref[...] = jnp.zeros((1, 1), jnp.float32)

    ce_ref[...] += ce_blk


def _sc_body(words_hbm, out_hbm, hist_v, buf0, buf1, sem0, sem1):
    cid = lax.axis_index("c")
    sid = lax.axis_index("s")
    wid = sid * 2 + cid
    wbase = wid * WPW
    c0 = lax.shift_right_logical(wid * 9, 4)
    bound = (c0 + 1) * NH

    zeros = jnp.zeros((16,), jnp.int32)
    ones = jnp.ones((16,), jnp.int32)

    def zbody(i, carry):
        hist_v[pl.ds(i * 16, 16)] = zeros
        return carry

    lax.fori_loop(0, HIST // 16, zbody, 0, unroll=8)

    def chunk_src(j):
        return words_hbm.at[pl.ds(wbase + j * CHUNK, CHUNK)]

    def process(j, buf):
        start = wbase + j * CHUNK
        off = (start >= bound).astype(jnp.int32) * (2 * K)

        def vec(i, c2):
            w = buf[pl.ds(i * 16, 16)]
            lo = (w & 0xFFFF) + off
            hi = lax.shift_right_logical(w, 16) + off
            plsc.addupdate_scatter(hist_v, [lo], ones)
            plsc.addupdate_scatter(hist_v, [hi], ones)
            return c2

        lax.fori_loop(0, CHUNK // 16, vec, 0, unroll=8)

    pltpu.async_copy(chunk_src(0), buf0, sem0)

    def pair(jp, carry):
        j0 = 2 * jp
        pltpu.async_copy(chunk_src(j0 + 1), buf1, sem1)
        pltpu.make_async_copy(chunk_src(j0), buf0, sem0).wait()
        process(j0, buf0)

        @pl.when(j0 + 2 < NCHUNK)
        def _():
            pltpu.async_copy(chunk_src(j0 + 2), buf0, sem0)

        pltpu.make_async_copy(chunk_src(j0 + 1), buf1, sem1).wait()
        process(j0 + 1, buf1)
        return carry

    lax.fori_loop(0, NCHUNK // 2, pair, 0)
    pltpu.sync_copy(hist_v, out_hbm.at[pl.ds(wid * HIST, HIST)])


def _suffix_sum(x):
    d = 1
    while d < K:
        pad = jnp.zeros((C, d), jnp.float32)
        x = x + jnp.concatenate([x[:, d:], pad], axis=1)
        d *= 2
    return x


def _pass2_body(h_ref, mm_ref, ce_ref, tot_ref, ceo_ref, lov_ref):
    hf = h_ref[...].astype(jnp.float32)
    mm = mm_ref[...]
    merged = jnp.dot(mm, hf, preferred_element_type=jnp.float32)
    fh = merged[:, K:]
    mh = merged[:, :K] + fh
    sm = _suffix_sum(mh)
    sf = _suffix_sum(fh)
    n = sf[:, 0:1]
    s_abv = sm - mh
    f_lo = sf - fh
    j_hi = 1.0 - (n - sf) / (n + s_abv + mh - sf + 1e-6)
    j_lo = 1.0 - (n - f_lo) / (n + s_abv - f_lo + 1e-6)
    j_lo = jnp.where(s_abv > 0, j_lo, 0.0)
    kk = lax.broadcasted_iota(jnp.int32, (C, K), 1).astype(jnp.float32)
    e_hat = (kk + 0.5) * (1.0 / K)
    contrib = jnp.where(mh > 0, e_hat * (j_hi - j_lo), 0.0)
    losses = jnp.sum(contrib, axis=1, keepdims=True)
    pres = (n > 0).astype(jnp.float32)
    npres = jnp.sum(pres)
    lov = jnp.sum(losses * pres) / jnp.maximum(npres, 1.0)
    lov = jnp.where(npres > 0, lov, 0.0)
    ce = ce_ref[0, 0] * (1.0 / N)
    tot_ref[...] = jnp.full((1, 1), ce + 0.5 * lov, jnp.float32)
    ceo_ref[...] = jnp.full((1, 1), ce, jnp.float32)
    lov_ref[...] = jnp.full((1, 1), lov, jnp.float32)


def _merge_matrix():
    m = np.zeros((C, 2 * NW), np.float32)
    for w in range(NW):
        c0 = (9 * w) >> 4
        m[c0, 2 * w] = 1.0
        if c0 + 1 < C:
            m[c0 + 1, 2 * w + 1] = 1.0
    return jnp.asarray(m)


_pass1 = pl.pallas_call(
    _pass1_body,
    grid=(NBLK,),
    in_specs=[
        pl.BlockSpec((C, VB), lambda i: (0, i)),
        pl.BlockSpec((1, 1, VB), lambda i: (i, 0, 0)),
    ],
    out_specs=[
        pl.BlockSpec((C, HB), lambda i: (0, i)),
        pl.BlockSpec((1, 1), lambda i: (0, 0)),
    ],
    out_shape=[
        jax.ShapeDtypeStruct((C, NH), jnp.int32),
        jax.ShapeDtypeStruct((1, 1), jnp.float32),
    ],
)


@functools.cache
def _sc_hist():
    return pl.kernel(
        _sc_body,
        out_type=jax.ShapeDtypeStruct((NW * HIST,), jnp.int32),
        mesh=plsc.VectorSubcoreMesh(
            core_axis_name="c", subcore_axis_name="s", num_cores=2),
        scratch_types=[
            pltpu.VMEM((HIST,), jnp.int32),
            pltpu.VMEM((CHUNK,), jnp.int32),
            pltpu.VMEM((CHUNK,), jnp.int32),
            pltpu.SemaphoreType.DMA,
            pltpu.SemaphoreType.DMA,
        ],
        compiler_params=pltpu.CompilerParams(needs_layout_passes=False),
    )


_pass2 = pl.pallas_call(
    _pass2_body,
    out_shape=[
        jax.ShapeDtypeStruct((1, 1), jnp.float32),
        jax.ShapeDtypeStruct((1, 1), jnp.float32),
        jax.ShapeDtypeStruct((1, 1), jnp.float32),
    ],
)


def kernel(semantic, sem_target):
    sem2 = jnp.transpose(semantic, (0, 1, 2, 4, 3)).reshape(C, N)
    tgt3 = jnp.transpose(sem_target, (0, 1, 3, 2)).reshape(NBLK, 1, VB)
    tgt3 = tgt3.astype(jnp.int32)
    words, ce = _pass1(sem2, tgt3)
    hist = _sc_hist()(words.reshape(C * NH))
    hf = hist.reshape(2 * NW, 2 * K)
    tot, ceo, lov = _pass2(hf, _merge_matrix(), ce)
    return tot.reshape(()), ceo.reshape(()), lov.reshape(())


# speedup vs baseline: 106.2900x; 1.8923x over previous
import functools

import numpy as np
import jax
import jax.numpy as jnp
from jax import lax
from jax.experimental import pallas as pl
from jax.experimental.pallas import tpu as pltpu
from jax.experimental.pallas import tpu_sc as plsc

C = 18
YD = 200
XZ = 3200
N = XZ * YD
K = 8192
HIST = 4 * K
NW = 32
SB = 128
NBLK = XZ // SB
PR = XZ // 2
NST = C * (PR // 8)
SPC = PR // 8
BR = 64
NVEC = YD // 16


def _pass1_body(sem_ref, tgt_ref, words_ref, ce_ref):
    l = sem_ref[...]
    t = tgt_ref[...][None]
    m = jnp.max(l, axis=0, keepdims=True)
    ex = jnp.exp(l - m)
    s = jnp.sum(ex, axis=0, keepdims=True)
    p = ex / s
    ci = lax.broadcasted_iota(jnp.int32, (C, SB, YD), 0)
    fg = ci == t
    err = jnp.where(fg, 1.0 - p, p)
    qa = jnp.minimum((err * K).astype(jnp.int32), K - 1)
    key = jnp.where(fg, qa + K, qa)
    ka = key[:, :SB // 2, :]
    kb = key[:, SB // 2:, :]
    words_ref[...] = ka | (kb << 16)
    lse = m + jnp.log(s)
    lt = jnp.sum(jnp.where(fg, l, 0.0), axis=0, keepdims=True)
    ce_blk = jnp.sum(lse - lt)

    @pl.when(pl.program_id(0) == 0)
    def _():
        ce_ref[...] = jnp.zeros((1, 1), jnp.float32)

    ce_ref[...] += ce_blk


def _sc_body(words_hbm, out_hbm, hist_v, buf_v):
    cid = lax.axis_index("c")
    sid = lax.axis_index("s")
    wid = sid * 2 + cid
    t0 = lax.shift_right_logical(wid * 225, 1)
    t1 = lax.shift_right_logical((wid + 1) * 225, 1)
    c0 = lax.shift_right_logical(wid * 9, 4)
    bnd = (c0 + 1) * SPC

    zeros = jnp.zeros((16,), jnp.int32)
    ones = jnp.ones((16,), jnp.int32)
    tailmask = lax.iota(jnp.int32, 16) >= 8

    def zbody(i, carry):
        hist_v[pl.ds(i * 16, 16)] = zeros
        return carry

    lax.fori_loop(0, HIST // 16, zbody, 0, unroll=8)

    def make_rows(off):
        def rows(nrows):
            def row(r, c2):
                def vfull(k, c3):
                    w = buf_v[r, pl.ds(k * 16, 16)]
                    lo = (w & 0xFFFF) + off
                    hi = lax.shift_right_logical(w, 16) + off
                    plsc.addupdate_scatter(hist_v, [lo], ones)
                    plsc.addupdate_scatter(hist_v, [hi], ones)
                    return c3

                lax.fori_loop(0, NVEC, vfull, 0, unroll=6)
                w = buf_v[r, pl.ds(YD - 16, 16)]
                lo = (w & 0xFFFF) + off
                hi = lax.shift_right_logical(w, 16) + off
                plsc.addupdate_scatter(hist_v, [lo], ones, mask=tailmask)
                plsc.addupdate_scatter(hist_v, [hi], ones, mask=tailmask)
                return c2

            lax.fori_loop(0, nrows, row, 0)

        return rows

    def do_range(ta, tb, off, cls):
        rows = make_rows(off)
        n = jnp.maximum(tb - ta, 0)
        nb = n // (BR // 8)

        def batch(b, carry):
            st = ta + (BR // 8) * b
            row0 = (st - cls * SPC) * 8
            pltpu.sync_copy(
                words_hbm.at[cls, pl.ds(row0, BR), pl.ds(0, YD)], buf_v)
            rows(BR)
            return carry

        lax.fori_loop(0, nb, batch, 0)

        def rem(i, carry):
            st = ta + (BR // 8) * nb + i
            row0 = (st - cls * SPC) * 8
            pltpu.sync_copy(
                words_hbm.at[cls, pl.ds(row0, 8), pl.ds(0, YD)],
                buf_v.at[pl.ds(0, 8)])
            rows(8)
            return carry

        lax.fori_loop(0, n - nb * (BR // 8), rem, 0)

    tsplit = jnp.clip(bnd, t0, t1)
    do_range(t0, tsplit, 0, c0)
    do_range(tsplit, t1, 2 * K, c0 + 1)
    pltpu.sync_copy(hist_v, out_hbm.at[pl.ds(wid * HIST, HIST)])


def _suffix_sum(x):
    d = 1
    while d < K:
        pad = jnp.zeros((C, d), jnp.float32)
        x = x + jnp.concatenate([x[:, d:], pad], axis=1)
        d *= 2
    return x


def _pass2_body(h_ref, mm_ref, ce_ref, tot_ref, ceo_ref, lov_ref):
    hf = h_ref[...].astype(jnp.float32)
    mm = mm_ref[...]
    merged = jnp.dot(mm, hf, preferred_element_type=jnp.float32)
    fh = merged[:, K:]
    mh = merged[:, :K] + fh
    sm = _suffix_sum(mh)
    sf = _suffix_sum(fh)
    n = sf[:, 0:1]
    s_abv = sm - mh
    f_lo = sf - fh
    j_hi = 1.0 - (n - sf) / (n + s_abv + mh - sf + 1e-6)
    j_lo = 1.0 - (n - f_lo) / (n + s_abv - f_lo + 1e-6)
    j_lo = jnp.where(s_abv > 0, j_lo, 0.0)
    kk = lax.broadcasted_iota(jnp.int32, (C, K), 1).astype(jnp.float32)
    e_hat = (kk + 0.5) * (1.0 / K)
    contrib = jnp.where(mh > 0, e_hat * (j_hi - j_lo), 0.0)
    losses = jnp.sum(contrib, axis=1, keepdims=True)
    pres = (n > 0).astype(jnp.float32)
    npres = jnp.sum(pres)
    lov = jnp.sum(losses * pres) / jnp.maximum(npres, 1.0)
    lov = jnp.where(npres > 0, lov, 0.0)
    ce = ce_ref[0, 0] * (1.0 / N)
    tot_ref[...] = jnp.full((1, 1), ce + 0.5 * lov, jnp.float32)
    ceo_ref[...] = jnp.full((1, 1), ce, jnp.float32)
    lov_ref[...] = jnp.full((1, 1), lov, jnp.float32)


def _merge_matrix():
    m = np.zeros((C, 2 * NW), np.float32)
    for w in range(NW):
        c0 = (9 * w) >> 4
        m[c0, 2 * w] = 1.0
        if c0 + 1 < C:
            m[c0 + 1, 2 * w + 1] = 1.0
    return jnp.asarray(m)


_pass1 = pl.pallas_call(
    _pass1_body,
    grid=(NBLK,),
    in_specs=[
        pl.BlockSpec((C, SB, YD), lambda i: (0, i, 0)),
        pl.BlockSpec((SB, YD), lambda i: (i, 0)),
    ],
    out_specs=[
        pl.BlockSpec((C, SB // 2, YD), lambda i: (0, i, 0)),
        pl.BlockSpec((1, 1), lambda i: (0, 0)),
    ],
    out_shape=[
        jax.ShapeDtypeStruct((C, PR, YD), jnp.int32),
        jax.ShapeDtypeStruct((1, 1), jnp.float32),
    ],
)


@functools.cache
def _sc_hist():
    return pl.kernel(
        _sc_body,
        out_type=jax.ShapeDtypeStruct((NW * HIST,), jnp.int32),
        mesh=plsc.VectorSubcoreMesh(
            core_axis_name="c", subcore_axis_name="s", num_cores=2),
        scratch_types=[
            pltpu.VMEM((HIST,), jnp.int32),
            pltpu.VMEM((BR, YD), jnp.int32),
        ],
        compiler_params=pltpu.CompilerParams(needs_layout_passes=False),
    )


_pass2 = pl.pallas_call(
    _pass2_body,
    out_shape=[
        jax.ShapeDtypeStruct((1, 1), jnp.float32),
        jax.ShapeDtypeStruct((1, 1), jnp.float32),
        jax.ShapeDtypeStruct((1, 1), jnp.float32),
    ],
)


def kernel(semantic, sem_target):
    semv = jnp.transpose(semantic, (0, 1, 2, 4, 3)).reshape(C, XZ, YD)
    tgtv = jnp.transpose(sem_target, (0, 1, 3, 2)).reshape(XZ, YD)
    words, ce = _pass1(semv, tgtv.astype(jnp.int32))
    hist = _sc_hist()(words)
    hf = hist.reshape(2 * NW, 2 * K)
    tot, ceo, lov = _pass2(hf, _merge_matrix(), ce)
    return tot.reshape(()), ceo.reshape(()), lov.reshape(())


# speedup vs baseline: 111.7624x; 1.0515x over previous
import functools

import numpy as np
import jax
import jax.numpy as jnp
from jax import lax
from jax.experimental import pallas as pl
from jax.experimental.pallas import tpu as pltpu
from jax.experimental.pallas import tpu_sc as plsc

C = 18
YD = 200
XZ = 3200
N = XZ * YD
K = 8192
HIST = 4 * K
NW = 32
SB = 128
NBLK = XZ // SB
PR = XZ // 2
NST = C * (PR // 8)
SPC = PR // 8
BR = 64
NVEC = YD // 16


def _pass1_body(sem_ref, tgt_ref, words_ref, ce_ref):
    l = sem_ref[...]
    t = tgt_ref[...][None]
    m = jnp.max(l, axis=0, keepdims=True)
    ex = jnp.exp(l - m)
    s = jnp.sum(ex, axis=0, keepdims=True)
    p = ex / s
    ci = lax.broadcasted_iota(jnp.int32, (C, SB, YD), 0)
    fg = ci == t
    err = jnp.where(fg, 1.0 - p, p)
    qa = jnp.minimum((err * K).astype(jnp.int32), K - 1)
    key = jnp.where(fg, qa + K, qa)
    ka = key[:, :SB // 2, :]
    kb = key[:, SB // 2:, :]
    words_ref[...] = ka | (kb << 16)
    lse = m + jnp.log(s)
    lt = jnp.sum(jnp.where(fg, l, 0.0), axis=0, keepdims=True)
    ce_blk = jnp.sum(lse - lt)

    @pl.when(pl.program_id(0) == 0)
    def _():
        ce_ref[...] = jnp.zeros((1, 1), jnp.float32)

    ce_ref[...] += ce_blk


def _sc_body(words_hbm, out_hbm, hist_v, buf_v):
    cid = lax.axis_index("c")
    sid = lax.axis_index("s")
    wid = sid * 2 + cid
    t0 = lax.shift_right_logical(wid * 225, 1)
    t1 = lax.shift_right_logical((wid + 1) * 225, 1)
    c0 = lax.shift_right_logical(wid * 9, 4)
    bnd = (c0 + 1) * SPC

    zeros = jnp.zeros((16,), jnp.int32)
    ones = jnp.ones((16,), jnp.int32)
    tailmask = lax.iota(jnp.int32, 16) >= 8

    def zbody(i, carry):
        hist_v[pl.ds(i * 16, 16)] = zeros
        return carry

    lax.fori_loop(0, HIST // 16, zbody, 0, unroll=8)

    def make_rows(off):
        hview = hist_v.at[pl.ds(off, 2 * K)]

        def rows(nrows):
            def row(r, c2):
                def vfull(k, c3):
                    w = buf_v[r, pl.ds(k * 16, 16)]
                    lo = w & 0xFFFF
                    hi = lax.shift_right_logical(w, 16)
                    plsc.addupdate_scatter(hview, [lo], ones)
                    plsc.addupdate_scatter(hview, [hi], ones)
                    return c3

                lax.fori_loop(0, NVEC, vfull, 0, unroll=6)
                w = buf_v[r, pl.ds(YD - 16, 16)]
                lo = w & 0xFFFF
                hi = lax.shift_right_logical(w, 16)
                plsc.addupdate_scatter(hview, [lo], ones, mask=tailmask)
                plsc.addupdate_scatter(hview, [hi], ones, mask=tailmask)
                return c2

            lax.fori_loop(0, nrows, row, 0)

        return rows

    def do_range(ta, tb, off, cls):
        rows = make_rows(off)
        n = jnp.maximum(tb - ta, 0)
        nb = n // (BR // 8)

        def batch(b, carry):
            st = ta + (BR // 8) * b
            row0 = (st - cls * SPC) * 8
            pltpu.sync_copy(
                words_hbm.at[cls, pl.ds(row0, BR), pl.ds(0, YD)], buf_v)
            rows(BR)
            return carry

        lax.fori_loop(0, nb, batch, 0)

        def rem(i, carry):
            st = ta + (BR // 8) * nb + i
            row0 = (st - cls * SPC) * 8
            pltpu.sync_copy(
                words_hbm.at[cls, pl.ds(row0, 8), pl.ds(0, YD)],
                buf_v.at[pl.ds(0, 8)])
            rows(8)
            return carry

        lax.fori_loop(0, n - nb * (BR // 8), rem, 0)

    tsplit = jnp.clip(bnd, t0, t1)
    do_range(t0, tsplit, 0, c0)
    do_range(tsplit, t1, 2 * K, c0 + 1)
    pltpu.sync_copy(hist_v, out_hbm.at[pl.ds(wid * HIST, HIST)])


def _suffix_sum(x):
    d = 1
    while d < K:
        pad = jnp.zeros((C, d), jnp.float32)
        x = x + jnp.concatenate([x[:, d:], pad], axis=1)
        d *= 2
    return x


def _pass2_body(h_ref, mm_ref, ce_ref, tot_ref, ceo_ref, lov_ref):
    hf = h_ref[...].astype(jnp.float32)
    mm = mm_ref[...]
    merged = jnp.dot(mm, hf, preferred_element_type=jnp.float32)
    fh = merged[:, K:]
    mh = merged[:, :K] + fh
    sm = _suffix_sum(mh)
    sf = _suffix_sum(fh)
    n = sf[:, 0:1]
    s_abv = sm - mh
    f_lo = sf - fh
    j_hi = 1.0 - (n - sf) / (n + s_abv + mh - sf + 1e-6)
    j_lo = 1.0 - (n - f_lo) / (n + s_abv - f_lo + 1e-6)
    j_lo = jnp.where(s_abv > 0, j_lo, 0.0)
    kk = lax.broadcasted_iota(jnp.int32, (C, K), 1).astype(jnp.float32)
    e_hat = (kk + 0.5) * (1.0 / K)
    contrib = jnp.where(mh > 0, e_hat * (j_hi - j_lo), 0.0)
    losses = jnp.sum(contrib, axis=1, keepdims=True)
    pres = (n > 0).astype(jnp.float32)
    npres = jnp.sum(pres)
    lov = jnp.sum(losses * pres) / jnp.maximum(npres, 1.0)
    lov = jnp.where(npres > 0, lov, 0.0)
    ce = ce_ref[0, 0] * (1.0 / N)
    tot_ref[...] = jnp.full((1, 1), ce + 0.5 * lov, jnp.float32)
    ceo_ref[...] = jnp.full((1, 1), ce, jnp.float32)
    lov_ref[...] = jnp.full((1, 1), lov, jnp.float32)


def _merge_matrix():
    m = np.zeros((C, 2 * NW), np.float32)
    for w in range(NW):
        c0 = (9 * w) >> 4
        m[c0, 2 * w] = 1.0
        if c0 + 1 < C:
            m[c0 + 1, 2 * w + 1] = 1.0
    return jnp.asarray(m)


_pass1 = pl.pallas_call(
    _pass1_body,
    grid=(NBLK,),
    in_specs=[
        pl.BlockSpec((C, SB, YD), lambda i: (0, i, 0)),
        pl.BlockSpec((SB, YD), lambda i: (i, 0)),
    ],
    out_specs=[
        pl.BlockSpec((C, SB // 2, YD), lambda i: (0, i, 0)),
        pl.BlockSpec((1, 1), lambda i: (0, 0)),
    ],
    out_shape=[
        jax.ShapeDtypeStruct((C, PR, YD), jnp.int32),
        jax.ShapeDtypeStruct((1, 1), jnp.float32),
    ],
)


@functools.cache
def _sc_hist():
    return pl.kernel(
        _sc_body,
        out_type=jax.ShapeDtypeStruct((NW * HIST,), jnp.int32),
        mesh=plsc.VectorSubcoreMesh(
            core_axis_name="c", subcore_axis_name="s", num_cores=2),
        scratch_types=[
            pltpu.VMEM((HIST,), jnp.int32),
            pltpu.VMEM((BR, YD), jnp.int32),
        ],
        compiler_params=pltpu.CompilerParams(needs_layout_passes=False),
    )


_pass2 = pl.pallas_call(
    _pass2_body,
    out_shape=[
        jax.ShapeDtypeStruct((1, 1), jnp.float32),
        jax.ShapeDtypeStruct((1, 1), jnp.float32),
        jax.ShapeDtypeStruct((1, 1), jnp.float32),
    ],
)


def kernel(semantic, sem_target):
    semv = jnp.transpose(semantic, (0, 1, 2, 4, 3)).reshape(C, XZ, YD)
    tgtv = jnp.transpose(sem_target, (0, 1, 3, 2)).reshape(XZ, YD)
    words, ce = _pass1(semv, tgtv.astype(jnp.int32))
    hist = _sc_hist()(words)
    hf = hist.reshape(2 * NW, 2 * K)
    tot, ceo, lov = _pass2(hf, _merge_matrix(), ce)
    return tot.reshape(()), ceo.reshape(()), lov.reshape(())


# speedup vs baseline: 111.9942x; 1.0021x over previous
import functools

import numpy as np
import jax
import jax.numpy as jnp
from jax import lax
from jax.experimental import pallas as pl
from jax.experimental.pallas import tpu as pltpu
from jax.experimental.pallas import tpu_sc as plsc

C = 18
YD = 200
XZ = 3200
N = XZ * YD
K = 8192
HIST = 4 * K
NW = 32
SB = 128
NBLK = XZ // SB
PR = XZ // 2
NST = C * (PR // 8)
SPC = PR // 8
BR = 64
NVEC = YD // 16


def _pass1_body(sem_ref, tgt_ref, words_ref, ce_ref):
    l = sem_ref[...]
    t = tgt_ref[...][None]
    m = jnp.max(l, axis=0, keepdims=True)
    ex = jnp.exp(l - m)
    s = jnp.sum(ex, axis=0, keepdims=True)
    p = ex / s
    ci = lax.broadcasted_iota(jnp.int32, (C, SB, YD), 0)
    fg = ci == t
    err = jnp.where(fg, 1.0 - p, p)
    qa = jnp.minimum((err * K).astype(jnp.int32), K - 1)
    key = jnp.where(fg, qa + K, qa)
    ka = key[:, :SB // 2, :]
    kb = key[:, SB // 2:, :]
    words_ref[...] = ka | (kb << 16)
    lse = m + jnp.log(s)
    lt = jnp.sum(jnp.where(fg, l, 0.0), axis=0, keepdims=True)
    ce_blk = jnp.sum(lse - lt)

    @pl.when(pl.program_id(0) == 0)
    def _():
        ce_ref[...] = jnp.zeros((1, 1), jnp.float32)

    ce_ref[...] += ce_blk


def _sc_body(words_hbm, out_hbm, hist_v, buf_v):
    cid = lax.axis_index("c")
    sid = lax.axis_index("s")
    wid = sid * 2 + cid
    t0 = lax.shift_right_logical(wid * 225, 1)
    t1 = lax.shift_right_logical((wid + 1) * 225, 1)
    c0 = lax.shift_right_logical(wid * 9, 4)
    bnd = (c0 + 1) * SPC

    zeros = jnp.zeros((16,), jnp.int32)
    ones = jnp.ones((16,), jnp.int32)
    tailmask = lax.iota(jnp.int32, 16) >= 8

    def zbody(i, carry):
        hist_v[pl.ds(i * 16, 16)] = zeros
        return carry

    lax.fori_loop(0, HIST // 16, zbody, 0, unroll=8)

    def make_rows(off):
        hview = hist_v.at[pl.ds(off, 2 * K)]

        def rows(nrows):
            def row(r, c2):
                def vfull(k, c3):
                    w = buf_v[r, pl.ds(k * 16, 16)]
                    lo = w & 0xFFFF
                    hi = lax.shift_right_logical(w, 16)
                    plsc.addupdate_scatter(hview, [lo], ones)
                    plsc.addupdate_scatter(hview, [hi], ones)
                    return c3

                lax.fori_loop(0, NVEC, vfull, 0, unroll=12)
                w = buf_v[r, pl.ds(YD - 16, 16)]
                lo = w & 0xFFFF
                hi = lax.shift_right_logical(w, 16)
                plsc.addupdate_scatter(hview, [lo], ones, mask=tailmask)
                plsc.addupdate_scatter(hview, [hi], ones, mask=tailmask)
                return c2

            lax.fori_loop(0, nrows, row, 0, unroll=2)

        return rows

    def do_range(ta, tb, off, cls):
        rows = make_rows(off)
        n = jnp.maximum(tb - ta, 0)
        nb = n // (BR // 8)

        def batch(b, carry):
            st = ta + (BR // 8) * b
            row0 = (st - cls * SPC) * 8
            pltpu.sync_copy(
                words_hbm.at[cls, pl.ds(row0, BR), pl.ds(0, YD)], buf_v)
            rows(BR)
            return carry

        lax.fori_loop(0, nb, batch, 0)

        def rem(i, carry):
            st = ta + (BR // 8) * nb + i
            row0 = (st - cls * SPC) * 8
            pltpu.sync_copy(
                words_hbm.at[cls, pl.ds(row0, 8), pl.ds(0, YD)],
                buf_v.at[pl.ds(0, 8)])
            rows(8)
            return carry

        lax.fori_loop(0, n - nb * (BR // 8), rem, 0)

    tsplit = jnp.clip(bnd, t0, t1)
    do_range(t0, tsplit, 0, c0)
    do_range(tsplit, t1, 2 * K, c0 + 1)
    pltpu.sync_copy(hist_v, out_hbm.at[pl.ds(wid * HIST, HIST)])


def _suffix_sum(x):
    d = 1
    while d < K:
        pad = jnp.zeros((C, d), jnp.float32)
        x = x + jnp.concatenate([x[:, d:], pad], axis=1)
        d *= 2
    return x


def _pass2_body(h_ref, mm_ref, ce_ref, tot_ref, ceo_ref, lov_ref):
    hf = h_ref[...].astype(jnp.float32)
    mm = mm_ref[...]
    merged = jnp.dot(mm, hf, preferred_element_type=jnp.float32)
    fh = merged[:, K:]
    mh = merged[:, :K] + fh
    sm = _suffix_sum(mh)
    sf = _suffix_sum(fh)
    n = sf[:, 0:1]
    s_abv = sm - mh
    f_lo = sf - fh
    j_hi = 1.0 - (n - sf) / (n + s_abv + mh - sf + 1e-6)
    j_lo = 1.0 - (n - f_lo) / (n + s_abv - f_lo + 1e-6)
    j_lo = jnp.where(s_abv > 0, j_lo, 0.0)
    kk = lax.broadcasted_iota(jnp.int32, (C, K), 1).astype(jnp.float32)
    e_hat = (kk + 0.5) * (1.0 / K)
    contrib = jnp.where(mh > 0, e_hat * (j_hi - j_lo), 0.0)
    losses = jnp.sum(contrib, axis=1, keepdims=True)
    pres = (n > 0).astype(jnp.float32)
    npres = jnp.sum(pres)
    lov = jnp.sum(losses * pres) / jnp.maximum(npres, 1.0)
    lov = jnp.where(npres > 0, lov, 0.0)
    ce = ce_ref[0, 0] * (1.0 / N)
    tot_ref[...] = jnp.full((1, 1), ce + 0.5 * lov, jnp.float32)
    ceo_ref[...] = jnp.full((1, 1), ce, jnp.float32)
    lov_ref[...] = jnp.full((1, 1), lov, jnp.float32)


def _merge_matrix():
    m = np.zeros((C, 2 * NW), np.float32)
    for w in range(NW):
        c0 = (9 * w) >> 4
        m[c0, 2 * w] = 1.0
        if c0 + 1 < C:
            m[c0 + 1, 2 * w + 1] = 1.0
    return jnp.asarray(m)


_pass1 = pl.pallas_call(
    _pass1_body,
    grid=(NBLK,),
    in_specs=[
        pl.BlockSpec((C, SB, YD), lambda i: (0, i, 0)),
        pl.BlockSpec((SB, YD), lambda i: (i, 0)),
    ],
    out_specs=[
        pl.BlockSpec((C, SB // 2, YD), lambda i: (0, i, 0)),
        pl.BlockSpec((1, 1), lambda i: (0, 0)),
    ],
    out_shape=[
        jax.ShapeDtypeStruct((C, PR, YD), jnp.int32),
        jax.ShapeDtypeStruct((1, 1), jnp.float32),
    ],
)


@functools.cache
def _sc_hist():
    return pl.kernel(
        _sc_body,
        out_type=jax.ShapeDtypeStruct((NW * HIST,), jnp.int32),
        mesh=plsc.VectorSubcoreMesh(
            core_axis_name="c", subcore_axis_name="s", num_cores=2),
        scratch_types=[
            pltpu.VMEM((HIST,), jnp.int32),
            pltpu.VMEM((BR, YD), jnp.int32),
        ],
        compiler_params=pltpu.CompilerParams(needs_layout_passes=False),
    )


_pass2 = pl.pallas_call(
    _pass2_body,
    out_shape=[
        jax.ShapeDtypeStruct((1, 1), jnp.float32),
        jax.ShapeDtypeStruct((1, 1), jnp.float32),
        jax.ShapeDtypeStruct((1, 1), jnp.float32),
    ],
)


def kernel(semantic, sem_target):
    semv = jnp.transpose(semantic, (0, 1, 2, 4, 3)).reshape(C, XZ, YD)
    tgtv = jnp.transpose(sem_target, (0, 1, 3, 2)).reshape(XZ, YD)
    words, ce = _pass1(semv, tgtv.astype(jnp.int32))
    hist = _sc_hist()(words)
    hf = hist.reshape(2 * NW, 2 * K)
    tot, ceo, lov = _pass2(hf, _merge_matrix(), ce)
    return tot.reshape(()), ceo.reshape(()), lov.reshape(())
